# baseline (speedup 1.0000x reference)
import functools

import jax
import jax.numpy as jnp
from jax import lax
from jax.experimental import pallas as pl
from jax.experimental.pallas import tpu as pltpu
from jax.experimental.pallas import tpu_sc as plsc

N = 100000
K = 16
F = 32
KP = 15
KPP = 16
EXT = 0.06
E = N * K

D = 40
NC = 2
NS = 16
NW = NC * NS
PER_W = E // NW
CH = 1000
ROUNDS = PER_W // (2 * CH)


def _sc_gather(tbl, nbr):
    mesh = plsc.VectorSubcoreMesh(core_axis_name="c", subcore_axis_name="s")

    @functools.partial(
        pl.kernel,
        mesh=mesh,
        out_type=jax.ShapeDtypeStruct((E, D), jnp.float32),
        scratch_types=[
            pltpu.VMEM((CH,), jnp.int32),
            pltpu.VMEM((CH,), jnp.int32),
            pltpu.VMEM((CH, D), jnp.float32),
            pltpu.VMEM((CH, D), jnp.float32),
            pltpu.SemaphoreType.DMA,
            pltpu.SemaphoreType.DMA,
            pltpu.SemaphoreType.DMA,
        ],
        compiler_params=pltpu.CompilerParams(use_tc_tiling_on_sc=False),
    )
    def k(tbl_hbm, nbr_hbm, out_hbm, idx0, idx1, buf0, buf1, semi, semg,
          semw):
        wid = lax.axis_index("s") * NC + lax.axis_index("c")

        def body(j, carry):
            base0 = wid * PER_W + (2 * j) * CH
            base1 = base0 + CH
            hi0 = pltpu.async_copy(nbr_hbm.at[pl.ds(base0, CH)], idx0, semi)
            hi1 = pltpu.async_copy(nbr_hbm.at[pl.ds(base1, CH)], idx1, semi)
            hi0.wait()
            hg0 = pltpu.async_copy(tbl_hbm.at[idx0], buf0, semg)
            hi1.wait()
            hg1 = pltpu.async_copy(tbl_hbm.at[idx1], buf1, semg)
            hg0.wait()
            hw0 = pltpu.async_copy(buf0, out_hbm.at[pl.ds(base0, CH)], semw)
            hg1.wait()
            hw1 = pltpu.async_copy(buf1, out_hbm.at[pl.ds(base1, CH)], semw)
            hw0.wait()
            hw1.wait()
            return carry

        lax.fori_loop(0, ROUNDS, body, 0)

    return k(tbl, nbr)


NB = 200
EB = NB * K
WTOT = KPP * F


def _tc_body(ge_ref, pos_ref, m6_ref, kp2_ref, r_ref, wc_ref, h_ref,
             ssum_ref, out_ref):
    xg = ge_ref[:, 0:F]
    pj = ge_ref[:, F:F + 3]
    posv = pos_ref[...]
    ssum = ssum_ref[...]
    posr = lax.dot_general(ssum, posv, (((0,), (0,)), ((), ())),
                           precision=lax.Precision.HIGHEST,
                           preferred_element_type=jnp.float32)
    r = pj - posr
    rf = jnp.concatenate([r * r, r], axis=1)
    d2 = jnp.dot(rf, m6_ref[0:6, :], precision=lax.Precision.HIGHEST,
                 preferred_element_type=jnp.float32) + kp2_ref[...]
    dist = jnp.sqrt(jnp.maximum(d2, 0.0) + 1e-12)
    infl = jnp.maximum(0.0, 1.0 - dist / EXT)
    zc = jnp.dot(xg, wc_ref[...], preferred_element_type=jnp.float32)
    ir = jnp.dot(infl, r_ref[...], preferred_element_type=jnp.float32)
    u = zc * ir
    kfz = lax.dot_general(ssum, u, (((1,), (0,)), ((), ())),
                          preferred_element_type=jnp.float32)
    out_ref[...] = jnp.dot(kfz, h_ref[...],
                           preferred_element_type=jnp.float32)


def _tc(ge, pos, m6, kp2, rmat, wcat, hmat, ssum):
    return pl.pallas_call(
        _tc_body,
        grid=(N // NB,),
        in_specs=[
            pl.BlockSpec((EB, D), lambda i: (i, 0)),
            pl.BlockSpec((NB, 3), lambda i: (i, 0)),
            pl.BlockSpec((8, KPP), lambda i: (0, 0)),
            pl.BlockSpec((1, KPP), lambda i: (0, 0)),
            pl.BlockSpec((KPP, WTOT), lambda i: (0, 0)),
            pl.BlockSpec((F, WTOT), lambda i: (0, 0)),
            pl.BlockSpec((WTOT, F), lambda i: (0, 0)),
            pl.BlockSpec((NB, EB), lambda i: (0, 0)),
        ],
        out_specs=pl.BlockSpec((NB, F), lambda i: (i, 0)),
        out_shape=jax.ShapeDtypeStruct((N, F), jnp.float32),
    )(ge, pos, m6, kp2, rmat, wcat, hmat, ssum)


def kernel(x, pos, neighbors, kernel_points, weights):
    nbr = neighbors.astype(jnp.int32).reshape(E)
    tbl = jnp.concatenate(
        [x, pos, jnp.zeros((N, D - F - 3), jnp.float32)], axis=1)
    g = _sc_gather(tbl, nbr)

    kpp = jnp.concatenate(
        [kernel_points, jnp.zeros((KPP - KP, 3), jnp.float32)], axis=0)
    m6 = jnp.zeros((8, KPP), jnp.float32)
    m6 = m6.at[0:3, :].set(jnp.ones((3, KPP), jnp.float32))
    m6 = m6.at[3:6, :].set(-2.0 * kpp.T)
    kp2 = jnp.sum(kpp * kpp, axis=1)[None, :]
    rmat = jnp.repeat(jnp.eye(KPP, dtype=jnp.float32), F, axis=1)
    wpad = jnp.concatenate(
        [weights, jnp.zeros((KPP - KP, F, F), jnp.float32)], axis=0)
    wcat = wpad.transpose(1, 0, 2).reshape(F, WTOT)
    hmat = jnp.tile(jnp.eye(F, dtype=jnp.float32), (KPP, 1))
    ssum = jnp.repeat(jnp.eye(NB, dtype=jnp.float32), K, axis=1)
    return _tc(g, pos, m6, kp2, rmat, wcat, hmat, ssum)


# speedup vs baseline: 2.5903x; 2.5903x over previous
import functools

import jax
import jax.numpy as jnp
from jax import lax
from jax.experimental import pallas as pl
from jax.experimental.pallas import tpu as pltpu
from jax.experimental.pallas import tpu_sc as plsc

N = 100000
K = 16
F = 32
KP = 15
KPP = 16
EXT = 0.06
E = N * K

D = 40
NC = 2
NS = 16
NW = NC * NS
PER_W = E // NW
CH = 1000
ROUNDS = PER_W // (2 * CH)


def _sc_gather(tbl, nbr):
    mesh = plsc.VectorSubcoreMesh(core_axis_name="c", subcore_axis_name="s")

    @functools.partial(
        pl.kernel,
        mesh=mesh,
        out_type=jax.ShapeDtypeStruct((E, D), jnp.float32),
        scratch_types=[
            pltpu.VMEM((CH,), jnp.int32),
            pltpu.VMEM((CH,), jnp.int32),
            pltpu.VMEM((CH, D), jnp.float32),
            pltpu.VMEM((CH, D), jnp.float32),
            pltpu.SemaphoreType.DMA,
            pltpu.SemaphoreType.DMA,
            pltpu.SemaphoreType.DMA,
        ],
        compiler_params=pltpu.CompilerParams(use_tc_tiling_on_sc=False),
    )
    def k(tbl_hbm, nbr_hbm, out_hbm, idx0, idx1, buf0, buf1, semi, semg,
          semw):
        wid = lax.axis_index("s") * NC + lax.axis_index("c")

        def body(j, carry):
            base0 = wid * PER_W + (2 * j) * CH
            base1 = base0 + CH
            hi0 = pltpu.async_copy(nbr_hbm.at[pl.ds(base0, CH)], idx0, semi)
            hi1 = pltpu.async_copy(nbr_hbm.at[pl.ds(base1, CH)], idx1, semi)
            hi0.wait()
            hg0 = pltpu.async_copy(tbl_hbm.at[idx0], buf0, semg)
            hi1.wait()
            hg1 = pltpu.async_copy(tbl_hbm.at[idx1], buf1, semg)
            hg0.wait()
            hw0 = pltpu.async_copy(buf0, out_hbm.at[pl.ds(base0, CH)], semw)
            hg1.wait()
            hw1 = pltpu.async_copy(buf1, out_hbm.at[pl.ds(base1, CH)], semw)
            hw0.wait()
            hw1.wait()
            return carry

        lax.fori_loop(0, ROUNDS, body, 0)

    return k(tbl, nbr)


NB = 200
EB = NB * K
WTOT = KPP * F


def _tc_body(ge_ref, pos_ref, kp_ref, r_ref, wc_ref, h_ref, out_ref):
    xg = ge_ref[:, 0:F]
    posv = pos_ref[...]
    posr = jnp.broadcast_to(posv[:, None, :], (NB, K, 3)).reshape(EB, 3)
    acc = None
    for c in range(3):
        rc = ge_ref[:, F + c:F + c + 1] - posr[:, c:c + 1]
        dc = rc - kp_ref[c:c + 1, :]
        acc = dc * dc if acc is None else acc + dc * dc
    dist = jnp.sqrt(acc + 1e-12)
    infl = jnp.maximum(0.0, 1.0 - dist / EXT)
    zc = jnp.dot(xg, wc_ref[...], preferred_element_type=jnp.float32)
    ir = jnp.dot(infl, r_ref[...], preferred_element_type=jnp.float32)
    u = zc * ir
    kfz = u.reshape(NB, K, WTOT).sum(axis=1)
    out_ref[...] = jnp.dot(kfz, h_ref[...],
                           preferred_element_type=jnp.float32)


def _tc(ge, pos, kp_pad, rmat, wcat, hmat):
    return pl.pallas_call(
        _tc_body,
        grid=(N // NB,),
        in_specs=[
            pl.BlockSpec((EB, D), lambda i: (i, 0)),
            pl.BlockSpec((NB, 3), lambda i: (i, 0)),
            pl.BlockSpec((8, KPP), lambda i: (0, 0)),
            pl.BlockSpec((KPP, WTOT), lambda i: (0, 0)),
            pl.BlockSpec((F, WTOT), lambda i: (0, 0)),
            pl.BlockSpec((WTOT, F), lambda i: (0, 0)),
        ],
        out_specs=pl.BlockSpec((NB, F), lambda i: (i, 0)),
        out_shape=jax.ShapeDtypeStruct((N, F), jnp.float32),
    )(ge, pos, kp_pad, rmat, wcat, hmat)


def kernel(x, pos, neighbors, kernel_points, weights):
    nbr = neighbors.astype(jnp.int32).reshape(E)
    tbl = jnp.concatenate(
        [x, pos, jnp.zeros((N, D - F - 3), jnp.float32)], axis=1)
    g = _sc_gather(tbl, nbr)

    kp_pad = jnp.zeros((8, KPP), jnp.float32).at[0:3, 0:KP].set(
        kernel_points.T)
    rmat = jnp.repeat(jnp.eye(KPP, dtype=jnp.float32), F, axis=1)
    wpad = jnp.concatenate(
        [weights, jnp.zeros((KPP - KP, F, F), jnp.float32)], axis=0)
    wcat = wpad.transpose(1, 0, 2).reshape(F, WTOT)
    hmat = jnp.tile(jnp.eye(F, dtype=jnp.float32), (KPP, 1))
    return _tc(g, pos, kp_pad, rmat, wcat, hmat)


# speedup vs baseline: 2.6231x; 1.0127x over previous
import functools

import jax
import jax.numpy as jnp
from jax import lax
from jax.experimental import pallas as pl
from jax.experimental.pallas import tpu as pltpu
from jax.experimental.pallas import tpu_sc as plsc

N = 100000
K = 16
F = 32
KP = 15
KPP = 16
EXT = 0.06
E = N * K

D = 40
NC = 2
NS = 16
NW = NC * NS
EH = E // 2
PER_W = EH // NW
CH = 1000
ROUNDS = PER_W // (2 * CH)


def _sc_gather(tbl, nbr, h):
    mesh = plsc.VectorSubcoreMesh(core_axis_name="c", subcore_axis_name="s")

    @functools.partial(
        pl.kernel,
        mesh=mesh,
        out_type=jax.ShapeDtypeStruct((EH, D), jnp.float32),
        scratch_types=[
            pltpu.VMEM((CH,), jnp.int32),
            pltpu.VMEM((CH,), jnp.int32),
            pltpu.VMEM((CH, D), jnp.float32),
            pltpu.VMEM((CH, D), jnp.float32),
            pltpu.SemaphoreType.DMA,
            pltpu.SemaphoreType.DMA,
            pltpu.SemaphoreType.DMA,
        ],
        compiler_params=pltpu.CompilerParams(use_tc_tiling_on_sc=False),
    )
    def k(tbl_hbm, nbr_hbm, out_hbm, idx0, idx1, buf0, buf1, semi, semg,
          semw):
        wid = lax.axis_index("s") * NC + lax.axis_index("c")

        def body(j, carry):
            base0 = wid * PER_W + (2 * j) * CH
            base1 = base0 + CH
            hi0 = pltpu.async_copy(
                nbr_hbm.at[pl.ds(h * EH + base0, CH)], idx0, semi)
            hi1 = pltpu.async_copy(
                nbr_hbm.at[pl.ds(h * EH + base1, CH)], idx1, semi)
            hi0.wait()
            hg0 = pltpu.async_copy(tbl_hbm.at[idx0], buf0, semg)
            hi1.wait()
            hg1 = pltpu.async_copy(tbl_hbm.at[idx1], buf1, semg)
            hg0.wait()
            hw0 = pltpu.async_copy(buf0, out_hbm.at[pl.ds(base0, CH)], semw)
            hg1.wait()
            hw1 = pltpu.async_copy(buf1, out_hbm.at[pl.ds(base1, CH)], semw)
            hw0.wait()
            hw1.wait()
            return carry

        lax.fori_loop(0, ROUNDS, body, 0)
        tbase = wid * PER_W + 2 * ROUNDS * CH
        pltpu.sync_copy(nbr_hbm.at[pl.ds(h * EH + tbase, CH)], idx0)
        pltpu.async_copy(tbl_hbm.at[idx0], buf0, semg).wait()
        pltpu.sync_copy(buf0, out_hbm.at[pl.ds(tbase, CH)])

    return k(tbl, nbr)


NH = N // 2


NB = 200
EB = NB * K
WTOT = KPP * F


def _tc_body(ge_ref, pos_ref, kp_ref, r_ref, wc_ref, h_ref, out_ref):
    xg = ge_ref[:, 0:F]
    posv = pos_ref[...]
    posr = jnp.broadcast_to(posv[:, None, :], (NB, K, 3)).reshape(EB, 3)
    acc = None
    for c in range(3):
        rc = ge_ref[:, F + c:F + c + 1] - posr[:, c:c + 1]
        dc = rc - kp_ref[c:c + 1, :]
        acc = dc * dc if acc is None else acc + dc * dc
    dist = jnp.sqrt(acc + 1e-12)
    infl = jnp.maximum(0.0, 1.0 - dist / EXT)
    zc = jnp.dot(xg, wc_ref[...], preferred_element_type=jnp.float32)
    ir = jnp.dot(infl, r_ref[...], preferred_element_type=jnp.float32)
    u = zc * ir
    kfz = u.reshape(NB, K, WTOT).sum(axis=1)
    out_ref[...] = jnp.dot(kfz, h_ref[...],
                           preferred_element_type=jnp.float32)


def _tc(ge, pos, kp_pad, rmat, wcat, hmat):
    return pl.pallas_call(
        _tc_body,
        grid=(NH // NB,),
        in_specs=[
            pl.BlockSpec((EB, D), lambda i: (i, 0)),
            pl.BlockSpec((NB, 3), lambda i: (i, 0)),
            pl.BlockSpec((8, KPP), lambda i: (0, 0)),
            pl.BlockSpec((KPP, WTOT), lambda i: (0, 0)),
            pl.BlockSpec((F, WTOT), lambda i: (0, 0)),
            pl.BlockSpec((WTOT, F), lambda i: (0, 0)),
        ],
        out_specs=pl.BlockSpec((NB, F), lambda i: (i, 0)),
        out_shape=jax.ShapeDtypeStruct((NH, F), jnp.float32),
    )(ge, pos, kp_pad, rmat, wcat, hmat)


def kernel(x, pos, neighbors, kernel_points, weights):
    nbr = neighbors.astype(jnp.int32).reshape(E)
    tbl = jnp.concatenate(
        [x, pos, jnp.zeros((N, D - F - 3), jnp.float32)], axis=1)
    g0 = _sc_gather(tbl, nbr, 0)
    g1 = _sc_gather(tbl, nbr, 1)

    kp_pad = jnp.zeros((8, KPP), jnp.float32).at[0:3, 0:KP].set(
        kernel_points.T)
    rmat = jnp.repeat(jnp.eye(KPP, dtype=jnp.float32), F, axis=1)
    wpad = jnp.concatenate(
        [weights, jnp.zeros((KPP - KP, F, F), jnp.float32)], axis=0)
    wcat = wpad.transpose(1, 0, 2).reshape(F, WTOT)
    hmat = jnp.tile(jnp.eye(F, dtype=jnp.float32), (KPP, 1))
    out0 = _tc(g0, pos[:NH], kp_pad, rmat, wcat, hmat)
    out1 = _tc(g1, pos[NH:], kp_pad, rmat, wcat, hmat)
    return jnp.concatenate([out0, out1], axis=0)


# speedup vs baseline: 2.6539x; 1.0117x over previous
import functools

import jax
import jax.numpy as jnp
from jax import lax
from jax.experimental import pallas as pl
from jax.experimental.pallas import tpu as pltpu
from jax.experimental.pallas import tpu_sc as plsc

N = 100000
K = 16
F = 32
KP = 15
KPP = 16
EXT = 0.06
E = N * K

D = 40
NC = 2
NS = 16
NW = NC * NS
EH = E // 2
PER_W = EH // NW
CH = 1000
ROUNDS = PER_W // (2 * CH)


def _sc_gather(tbl, nbr, h):
    mesh = plsc.VectorSubcoreMesh(core_axis_name="c", subcore_axis_name="s")

    @functools.partial(
        pl.kernel,
        mesh=mesh,
        out_type=jax.ShapeDtypeStruct((EH, D), jnp.float32),
        scratch_types=[
            pltpu.VMEM((CH,), jnp.int32),
            pltpu.VMEM((CH,), jnp.int32),
            pltpu.VMEM((CH, D), jnp.float32),
            pltpu.VMEM((CH, D), jnp.float32),
            pltpu.SemaphoreType.DMA,
            pltpu.SemaphoreType.DMA,
            pltpu.SemaphoreType.DMA,
        ],
        compiler_params=pltpu.CompilerParams(use_tc_tiling_on_sc=False),
    )
    def k(tbl_hbm, nbr_hbm, out_hbm, idx0, idx1, buf0, buf1, semi, semg,
          semw):
        wid = lax.axis_index("s") * NC + lax.axis_index("c")

        def body(j, carry):
            base0 = wid * PER_W + (2 * j) * CH
            base1 = base0 + CH
            hi0 = pltpu.async_copy(
                nbr_hbm.at[pl.ds(h * EH + base0, CH)], idx0, semi)
            hi1 = pltpu.async_copy(
                nbr_hbm.at[pl.ds(h * EH + base1, CH)], idx1, semi)
            hi0.wait()
            hg0 = pltpu.async_copy(tbl_hbm.at[idx0], buf0, semg)
            hi1.wait()
            hg1 = pltpu.async_copy(tbl_hbm.at[idx1], buf1, semg)
            hg0.wait()
            hw0 = pltpu.async_copy(buf0, out_hbm.at[pl.ds(base0, CH)], semw)
            hg1.wait()
            hw1 = pltpu.async_copy(buf1, out_hbm.at[pl.ds(base1, CH)], semw)
            hw0.wait()
            hw1.wait()
            return carry

        lax.fori_loop(0, ROUNDS, body, 0)
        tbase = wid * PER_W + 2 * ROUNDS * CH
        pltpu.sync_copy(nbr_hbm.at[pl.ds(h * EH + tbase, CH)], idx0)
        pltpu.async_copy(tbl_hbm.at[idx0], buf0, semg).wait()
        pltpu.sync_copy(buf0, out_hbm.at[pl.ds(tbase, CH)])

    return k(tbl, nbr)


NH = N // 2


NB = 400
EB = NB * K
WTOT = KPP * F


def _tc_body(ge_ref, pos_ref, kp_ref, r_ref, wc_ref, h_ref, out_ref):
    xg = ge_ref[:, 0:F]
    posv = pos_ref[...]
    posr = jnp.broadcast_to(posv[:, None, :], (NB, K, 3)).reshape(EB, 3)
    acc = None
    for c in range(3):
        rc = ge_ref[:, F + c:F + c + 1] - posr[:, c:c + 1]
        dc = rc - kp_ref[c:c + 1, :]
        acc = dc * dc if acc is None else acc + dc * dc
    dist = jnp.sqrt(acc + 1e-12)
    infl = jnp.maximum(0.0, 1.0 - dist / EXT)
    zc = jnp.dot(xg, wc_ref[...], preferred_element_type=jnp.float32)
    ir = jnp.dot(infl, r_ref[...], preferred_element_type=jnp.float32)
    u = zc * ir
    kfz = u.reshape(NB, K, WTOT).sum(axis=1)
    out_ref[...] = jnp.dot(kfz, h_ref[...],
                           preferred_element_type=jnp.float32)


def _tc(ge, pos, kp_pad, rmat, wcat, hmat):
    return pl.pallas_call(
        _tc_body,
        grid=(NH // NB,),
        in_specs=[
            pl.BlockSpec((EB, D), lambda i: (i, 0)),
            pl.BlockSpec((NB, 3), lambda i: (i, 0)),
            pl.BlockSpec((8, KPP), lambda i: (0, 0)),
            pl.BlockSpec((KPP, WTOT), lambda i: (0, 0)),
            pl.BlockSpec((F, WTOT), lambda i: (0, 0)),
            pl.BlockSpec((WTOT, F), lambda i: (0, 0)),
        ],
        out_specs=pl.BlockSpec((NB, F), lambda i: (i, 0)),
        out_shape=jax.ShapeDtypeStruct((NH, F), jnp.float32),
    )(ge, pos, kp_pad, rmat, wcat, hmat)


def kernel(x, pos, neighbors, kernel_points, weights):
    nbr = neighbors.astype(jnp.int32).reshape(E)
    tbl = jnp.concatenate(
        [x, pos, jnp.zeros((N, D - F - 3), jnp.float32)], axis=1)
    g0 = _sc_gather(tbl, nbr, 0)
    g1 = _sc_gather(tbl, nbr, 1)

    kp_pad = jnp.zeros((8, KPP), jnp.float32).at[0:3, 0:KP].set(
        kernel_points.T)
    rmat = jnp.repeat(jnp.eye(KPP, dtype=jnp.float32), F, axis=1)
    wpad = jnp.concatenate(
        [weights, jnp.zeros((KPP - KP, F, F), jnp.float32)], axis=0)
    wcat = wpad.transpose(1, 0, 2).reshape(F, WTOT)
    hmat = jnp.tile(jnp.eye(F, dtype=jnp.float32), (KPP, 1))
    out0 = _tc(g0, pos[:NH], kp_pad, rmat, wcat, hmat)
    out1 = _tc(g1, pos[NH:], kp_pad, rmat, wcat, hmat)
    return jnp.concatenate([out0, out1], axis=0)
